# baseline (speedup 1.0000x reference)
import jax
import jax.numpy as jnp
from jax.experimental import pallas as pl

_STRIDE = 8.0
_AW = (10.0, 16.0, 33.0)
_AH = (13.0, 30.0, 23.0)


def _yolo_body(bt_ref, conf_ref, cls_ref, xywh_ref, idx_ref, cf_ref):
    j = pl.program_id(1)
    c = conf_ref.shape[-1]
    col = j * c + jax.lax.broadcasted_iota(jnp.int32, (1, c), 1)
    a = col >> 12
    rem = col & 4095
    gy = (rem >> 6).astype(jnp.float32)
    gx = (rem & 63).astype(jnp.float32)

    b = bt_ref[0]
    px = (jax.nn.sigmoid(b[0:1, :]) + gx) * _STRIDE
    py = (jax.nn.sigmoid(b[1:2, :]) + gy) * _STRIDE
    aw = jnp.where(a == 0, _AW[0], jnp.where(a == 1, _AW[1], _AW[2]))
    ah = jnp.where(a == 0, _AH[0], jnp.where(a == 1, _AH[1], _AH[2]))
    pw = jnp.exp(b[2:3, :]) * aw
    ph = jnp.exp(b[3:4, :]) * ah
    xywh_ref[0] = jnp.concatenate([px, py, pw, ph], axis=0)

    logits = cls_ref[0]
    m = jnp.max(logits, axis=-1)
    idx_ref[0, 0] = jnp.argmax(logits, axis=-1).astype(jnp.int32)
    cf_ref[0, 0] = jax.nn.sigmoid(conf_ref[0, 0]) * jax.nn.sigmoid(m)


def kernel(bbox, conf, cls_logits, img_size):
    nB, nA, nH, nW, _ = bbox.shape
    n = nA * nH * nW
    nC = cls_logits.shape[-1]
    bt = bbox.reshape(nB, n, 4).transpose(0, 2, 1)
    conf2 = conf.reshape(nB, 1, n)
    cls2 = cls_logits.reshape(nB, n, nC)

    C = 2048
    grid = (nB, n // C)
    xywh_t, cls_idx, confs = pl.pallas_call(
        _yolo_body,
        grid=grid,
        in_specs=[
            pl.BlockSpec((1, 4, C), lambda i, j: (i, 0, j)),
            pl.BlockSpec((1, 1, C), lambda i, j: (i, 0, j)),
            pl.BlockSpec((1, C, nC), lambda i, j: (i, j, 0)),
        ],
        out_specs=[
            pl.BlockSpec((1, 4, C), lambda i, j: (i, 0, j)),
            pl.BlockSpec((1, 1, C), lambda i, j: (i, 0, j)),
            pl.BlockSpec((1, 1, C), lambda i, j: (i, 0, j)),
        ],
        out_shape=[
            jax.ShapeDtypeStruct((nB, 4, n), jnp.float32),
            jax.ShapeDtypeStruct((nB, 1, n), jnp.int32),
            jax.ShapeDtypeStruct((nB, 1, n), jnp.float32),
        ],
    )(bt, conf2, cls2)

    p_xywh = xywh_t.transpose(0, 2, 1)
    return (p_xywh, cls_idx.reshape(nB, n), confs.reshape(nB, n))
